# baseline (speedup 1.0000x reference)
import functools

import jax
import jax.numpy as jnp
from jax import lax
from jax.experimental import pallas as pl
from jax.experimental.pallas import tpu as pltpu
from jax.experimental.pallas import tpu_sc as plsc

_DESIRED_SIZE = 448.0
_ALPHA = 0.5
_PENALTY_WEIGHT = 0.015
_EPS = 1e-7

_NC = 2
_NS = 16
_NW = _NC * _NS
_L = 16

_B = 16
_R = 20000
_N_BOXES = _B * _R
_BPW = _N_BOXES // _NW
_GROUPS = _BPW // _L

_mesh = plsc.VectorSubcoreMesh(
    core_axis_name="c", subcore_axis_name="s", num_cores=_NC, num_subcores=_NS
)


@functools.partial(
    pl.kernel,
    out_type=jax.ShapeDtypeStruct((_NW, 8, _L), jnp.float32),
    mesh=_mesh,
    scratch_types=[
        pltpu.VMEM((_BPW,), jnp.float32),
        pltpu.VMEM((_BPW,), jnp.float32),
        pltpu.VMEM((_BPW,), jnp.float32),
        pltpu.VMEM((_BPW,), jnp.float32),
        pltpu.VMEM((_BPW,), jnp.float32),
        pltpu.VMEM((_BPW,), jnp.float32),
        pltpu.VMEM((_BPW,), jnp.float32),
        pltpu.VMEM((_BPW,), jnp.float32),
        pltpu.VMEM((8, _L), jnp.float32),
    ],
)
def _loss_partials(pred_hbm, tgt_hbm, out_hbm,
                   x1v, y1v, x2v, y2v, tx1v, ty1v, tx2v, ty2v, acc_v):
    wid = lax.axis_index("s") * _NC + lax.axis_index("c")
    b = wid // 2
    r0 = (wid % 2) * _BPW
    for c, dst in enumerate((x1v, y1v, x2v, y2v)):
        pltpu.sync_copy(pred_hbm.at[pl.ds((c * _B + b) * _R + r0, _BPW)], dst)
    for c, dst in enumerate((tx1v, ty1v, tx2v, ty2v)):
        pltpu.sync_copy(tgt_hbm.at[pl.ds((b * 4 + c) * _R + r0, _BPW)], dst)

    zeros = jnp.zeros((_L,), jnp.float32)
    init = (zeros, zeros, zeros, zeros, zeros, zeros)

    @plsc.parallel_loop(0, _GROUPS, 1, unroll=8, carry=init)
    def _acc(g, carry):
        a_d, a_sl, a_sq, a_p, a_n, a_e = carry
        o = g * _L
        x1 = x1v[pl.ds(o, _L)]
        y1 = y1v[pl.ds(o, _L)]
        x2 = x2v[pl.ds(o, _L)]
        y2 = y2v[pl.ds(o, _L)]
        tx1 = tx1v[pl.ds(o, _L)]
        ty1 = ty1v[pl.ds(o, _L)]
        tx2 = tx2v[pl.ds(o, _L)]
        ty2 = ty2v[pl.ds(o, _L)]

        pred_area = jnp.maximum(x2 - x1, 0.0) * jnp.maximum(y2 - y1, 0.0)
        tgt_area = jnp.maximum(tx2 - tx1, 0.0) * jnp.maximum(ty2 - ty1, 0.0)
        inter = jnp.maximum(jnp.minimum(x2, tx2) - jnp.maximum(x1, tx1), 0.0) * \
            jnp.maximum(jnp.minimum(y2, ty2) - jnp.maximum(y1, ty1), 0.0)
        union_e = pred_area + tgt_area - inter + _EPS
        sx = (x1 + x2) - (tx1 + tx2)
        sy = (y1 + y2) - (ty1 + ty2)
        cdist = 0.25 * (sx * sx + sy * sy)
        ew = jnp.maximum(x2, tx2) - jnp.minimum(x1, tx1)
        eh = jnp.maximum(y2, ty2) - jnp.minimum(y1, ty1)
        diag_e = ew * ew + eh * eh + _EPS
        a_d = a_d + (cdist * union_e - inter * diag_e) / (union_e * diag_e)

        for p, t in ((x1, tx1), (y1, ty1), (x2, tx2), (y2, ty2)):
            ad = jnp.abs(p - t)
            cc = jnp.minimum(ad, 1.0)
            a_sl = a_sl + (ad - cc)
            a_sq = a_sq + cc * cc

        xd = x1 - x2
        yd = y1 - y2
        a_p = a_p + jnp.maximum(xd, 0.0) + jnp.maximum(yd, 0.0)
        a_p = a_p + jnp.where(xd >= 1.0, 1.0, 0.0)
        a_p = a_p + jnp.where(yd >= 1.0, 1.0, 0.0)
        for v in (x1, y1, x2, y2):
            a_n = a_n + jnp.minimum(v, 0.0)
            a_e = a_e + jnp.maximum(v, _DESIRED_SIZE)

        return a_d, a_sl, a_sq, a_p, a_n, a_e

    a_d, a_sl, a_sq, a_p, a_n, a_e = _acc
    acc_v[0, :] = a_d
    acc_v[1, :] = a_sl
    acc_v[2, :] = a_sq
    acc_v[3, :] = a_p
    acc_v[4, :] = a_n
    acc_v[5, :] = a_e
    acc_v[6, :] = zeros
    acc_v[7, :] = zeros
    pltpu.sync_copy(acc_v, out_hbm.at[wid])


def kernel(pred_boxes, target_boxes):
    pred_cols = jnp.transpose(pred_boxes, (2, 0, 1)).reshape(-1)
    tgt_cols = jnp.transpose(target_boxes, (0, 2, 1)).reshape(-1)
    parts = _loss_partials(pred_cols, tgt_cols)
    s = jnp.sum(parts, axis=(0, 2))
    dl = 1.0 + s[0] / _N_BOXES
    sl = (s[1] + 0.5 * s[2]) / (_N_BOXES * 4)
    pen = (s[3] - s[4] + (s[5] - 4.0 * _DESIRED_SIZE * _N_BOXES)) / _DESIRED_SIZE
    return _ALPHA * dl + (1.0 - _ALPHA) * sl + _PENALTY_WEIGHT * pen


# speedup vs baseline: 1.2207x; 1.2207x over previous
import functools

import jax
import jax.numpy as jnp
from jax import lax
from jax.experimental import pallas as pl
from jax.experimental.pallas import tpu as pltpu
from jax.experimental.pallas import tpu_sc as plsc

_DESIRED_SIZE = 448.0
_ALPHA = 0.5
_PENALTY_WEIGHT = 0.015
_EPS = 1e-7

_NC = 2
_NS = 16
_NW = _NC * _NS
_L = 16

_B = 16
_R = 20000
_N_BOXES = _B * _R
_CM = 19968
_CPW = _CM // 16
_DMA_COLS = 1280
_GROUPS = 8 * (_CPW // _L)

_mesh = plsc.VectorSubcoreMesh(
    core_axis_name="c", subcore_axis_name="s", num_cores=_NC, num_subcores=_NS
)


@functools.partial(
    pl.kernel,
    out_type=jax.ShapeDtypeStruct((_NW, 8, _L), jnp.float32),
    mesh=_mesh,
    scratch_types=[
        pltpu.VMEM((8, _DMA_COLS), jnp.float32),
        pltpu.VMEM((8, _DMA_COLS), jnp.float32),
        pltpu.VMEM((8, _DMA_COLS), jnp.float32),
        pltpu.VMEM((8, _DMA_COLS), jnp.float32),
        pltpu.VMEM((8, 4, _DMA_COLS), jnp.float32),
        pltpu.VMEM((8, _L), jnp.float32),
    ],
    compiler_params=pltpu.CompilerParams(
        needs_layout_passes=False, use_tc_tiling_on_sc=True),
)
def _loss_partials(pred_hbm, tgt_hbm, out_hbm,
                   x1v, y1v, x2v, y2v, tgt_v, acc_v):
    wid = lax.axis_index("s") * _NC + lax.axis_index("c")
    o = wid % 2
    k = wid // 2
    c0 = k * _CPW
    o0 = (c0 // 128) * 128
    rel0 = c0 - o0
    for c, dst in enumerate((x1v, y1v, x2v, y2v)):
        pltpu.sync_copy(
            pred_hbm.at[c, pl.ds(8 * o, 8), pl.ds(o0, _DMA_COLS)], dst)
    for b in range(8):
        pltpu.sync_copy(
            tgt_hbm.at[8 * o + b, pl.ds(0, 4), pl.ds(o0, _DMA_COLS)],
            tgt_v.at[b])

    zeros = jnp.zeros((_L,), jnp.float32)
    init = (zeros, zeros, zeros, zeros, zeros, zeros)

    @plsc.parallel_loop(0, _GROUPS, 1, unroll=8, carry=init)
    def _acc(g, carry):
        a_d, a_sl, a_sq, a_p, a_n, a_e = carry
        b = g // 78
        col = rel0 + (g % 78) * _L
        x1 = x1v[b, pl.ds(col, _L)]
        y1 = y1v[b, pl.ds(col, _L)]
        x2 = x2v[b, pl.ds(col, _L)]
        y2 = y2v[b, pl.ds(col, _L)]
        tx1 = tgt_v[b, 0, pl.ds(col, _L)]
        ty1 = tgt_v[b, 1, pl.ds(col, _L)]
        tx2 = tgt_v[b, 2, pl.ds(col, _L)]
        ty2 = tgt_v[b, 3, pl.ds(col, _L)]

        pred_area = jnp.maximum(x2 - x1, 0.0) * jnp.maximum(y2 - y1, 0.0)
        tgt_area = jnp.maximum(tx2 - tx1, 0.0) * jnp.maximum(ty2 - ty1, 0.0)
        inter = jnp.maximum(jnp.minimum(x2, tx2) - jnp.maximum(x1, tx1), 0.0) * \
            jnp.maximum(jnp.minimum(y2, ty2) - jnp.maximum(y1, ty1), 0.0)
        union_e = pred_area + tgt_area - inter + _EPS
        sx = (x1 + x2) - (tx1 + tx2)
        sy = (y1 + y2) - (ty1 + ty2)
        cdist = 0.25 * (sx * sx + sy * sy)
        ew = jnp.maximum(x2, tx2) - jnp.minimum(x1, tx1)
        eh = jnp.maximum(y2, ty2) - jnp.minimum(y1, ty1)
        diag_e = ew * ew + eh * eh + _EPS
        a_d = a_d + (cdist * union_e - inter * diag_e) / (union_e * diag_e)

        for p, t in ((x1, tx1), (y1, ty1), (x2, tx2), (y2, ty2)):
            ad = jnp.abs(p - t)
            cc = jnp.minimum(ad, 1.0)
            a_sl = a_sl + (ad - cc)
            a_sq = a_sq + cc * cc

        xd = x1 - x2
        yd = y1 - y2
        a_p = a_p + jnp.maximum(xd, 0.0) + jnp.maximum(yd, 0.0)
        a_p = a_p + jnp.where(xd >= 1.0, 1.0, 0.0)
        a_p = a_p + jnp.where(yd >= 1.0, 1.0, 0.0)
        for v in (x1, y1, x2, y2):
            a_n = a_n + jnp.minimum(v, 0.0)
            a_e = a_e + jnp.maximum(v, _DESIRED_SIZE)

        return a_d, a_sl, a_sq, a_p, a_n, a_e

    a_d, a_sl, a_sq, a_p, a_n, a_e = _acc
    acc_v[0, :] = a_d
    acc_v[1, :] = a_sl
    acc_v[2, :] = a_sq
    acc_v[3, :] = a_p
    acc_v[4, :] = a_n
    acc_v[5, :] = a_e - (4.0 * _DESIRED_SIZE * _GROUPS)
    acc_v[6, :] = zeros
    acc_v[7, :] = zeros
    pltpu.sync_copy(acc_v, out_hbm.at[wid])


def _sums6(x1, y1, x2, y2, tx1, ty1, tx2, ty2):
    pred_area = jnp.maximum(x2 - x1, 0.0) * jnp.maximum(y2 - y1, 0.0)
    tgt_area = jnp.maximum(tx2 - tx1, 0.0) * jnp.maximum(ty2 - ty1, 0.0)
    inter = jnp.maximum(jnp.minimum(x2, tx2) - jnp.maximum(x1, tx1), 0.0) * \
        jnp.maximum(jnp.minimum(y2, ty2) - jnp.maximum(y1, ty1), 0.0)
    union_e = pred_area + tgt_area - inter + _EPS
    sx = (x1 + x2) - (tx1 + tx2)
    sy = (y1 + y2) - (ty1 + ty2)
    cdist = 0.25 * (sx * sx + sy * sy)
    ew = jnp.maximum(x2, tx2) - jnp.minimum(x1, tx1)
    eh = jnp.maximum(y2, ty2) - jnp.minimum(y1, ty1)
    diag_e = ew * ew + eh * eh + _EPS
    s0 = jnp.sum((cdist * union_e - inter * diag_e) / (union_e * diag_e))
    s1 = 0.0
    s2 = 0.0
    for p, t in ((x1, tx1), (y1, ty1), (x2, tx2), (y2, ty2)):
        ad = jnp.abs(p - t)
        cc = jnp.minimum(ad, 1.0)
        s1 = s1 + jnp.sum(ad - cc)
        s2 = s2 + jnp.sum(cc * cc)
    xd = x1 - x2
    yd = y1 - y2
    s3 = jnp.sum(jnp.maximum(xd, 0.0) + jnp.maximum(yd, 0.0)
                 + jnp.where(xd >= 1.0, 1.0, 0.0)
                 + jnp.where(yd >= 1.0, 1.0, 0.0))
    s4 = 0.0
    s5 = 0.0
    for v in (x1, y1, x2, y2):
        s4 = s4 + jnp.sum(jnp.minimum(v, 0.0))
        s5 = s5 + jnp.sum(jnp.maximum(v - _DESIRED_SIZE, 0.0))
    return jnp.stack([s0, s1, s2, s3, s4, s5, 0.0, 0.0])


def kernel(pred_boxes, target_boxes):
    pred_cols = jnp.transpose(pred_boxes, (2, 0, 1))
    tgt_cols = jnp.transpose(target_boxes, (0, 2, 1))
    parts = _loss_partials(pred_cols, tgt_cols)
    tp = pred_boxes[:, _CM:, :]
    tt = target_boxes[:, _CM:, :]
    tail = _sums6(tp[..., 0], tp[..., 1], tp[..., 2], tp[..., 3],
                  tt[..., 0], tt[..., 1], tt[..., 2], tt[..., 3])
    s = jnp.sum(parts, axis=(0, 2)) + tail
    dl = 1.0 + s[0] / _N_BOXES
    sl = (s[1] + 0.5 * s[2]) / (_N_BOXES * 4)
    pen = (s[3] - s[4] + s[5]) / _DESIRED_SIZE
    return _ALPHA * dl + (1.0 - _ALPHA) * sl + _PENALTY_WEIGHT * pen


# speedup vs baseline: 1.2248x; 1.0034x over previous
import functools

import jax
import jax.numpy as jnp
from jax import lax
from jax.experimental import pallas as pl
from jax.experimental.pallas import tpu as pltpu
from jax.experimental.pallas import tpu_sc as plsc

_DESIRED_SIZE = 448.0
_ALPHA = 0.5
_PENALTY_WEIGHT = 0.015
_EPS = 1e-7

_NC = 2
_NS = 16
_NW = _NC * _NS
_L = 16

_B = 16
_R = 20000
_N_BOXES = _B * _R
_CM = 19968
_CPW = _CM // 16
_DMA_COLS = 1280
_GROUPS = 8 * (_CPW // _L)

_mesh = plsc.VectorSubcoreMesh(
    core_axis_name="c", subcore_axis_name="s", num_cores=_NC, num_subcores=_NS
)


@functools.partial(
    pl.kernel,
    out_type=jax.ShapeDtypeStruct((_NW, 8, _L), jnp.float32),
    mesh=_mesh,
    scratch_types=[
        pltpu.VMEM((8, _DMA_COLS), jnp.float32),
        pltpu.VMEM((8, _DMA_COLS), jnp.float32),
        pltpu.VMEM((8, _DMA_COLS), jnp.float32),
        pltpu.VMEM((8, _DMA_COLS), jnp.float32),
        pltpu.VMEM((8, 4, _DMA_COLS), jnp.float32),
        pltpu.VMEM((8, _L), jnp.float32),
    ],
    compiler_params=pltpu.CompilerParams(
        needs_layout_passes=False, use_tc_tiling_on_sc=True),
)
def _loss_partials(pred_hbm, tgt_hbm, out_hbm,
                   x1v, y1v, x2v, y2v, tgt_v, acc_v):
    wid = lax.axis_index("s") * _NC + lax.axis_index("c")
    o = wid % 2
    k = wid // 2
    c0 = k * _CPW
    o0 = (c0 // 128) * 128
    rel0 = c0 - o0
    for c, dst in enumerate((x1v, y1v, x2v, y2v)):
        pltpu.sync_copy(
            pred_hbm.at[c, pl.ds(8 * o, 8), pl.ds(o0, _DMA_COLS)], dst)
    for b in range(8):
        pltpu.sync_copy(
            tgt_hbm.at[8 * o + b, pl.ds(0, 4), pl.ds(o0, _DMA_COLS)],
            tgt_v.at[b])

    zeros = jnp.zeros((_L,), jnp.float32)
    init = (zeros, zeros, zeros, zeros, zeros, zeros)

    @plsc.parallel_loop(0, _CPW // _L, 1, unroll=2, carry=init)
    def _acc(j, carry):
        a_d, a_sl, a_sq, a_p, a_n, a_e = carry
        col = rel0 + j * _L
        for b in range(8):
            x1 = x1v[b, pl.ds(col, _L)]
            y1 = y1v[b, pl.ds(col, _L)]
            x2 = x2v[b, pl.ds(col, _L)]
            y2 = y2v[b, pl.ds(col, _L)]
            tx1 = tgt_v[b, 0, pl.ds(col, _L)]
            ty1 = tgt_v[b, 1, pl.ds(col, _L)]
            tx2 = tgt_v[b, 2, pl.ds(col, _L)]
            ty2 = tgt_v[b, 3, pl.ds(col, _L)]

            pred_area = jnp.maximum(x2 - x1, 0.0) * jnp.maximum(y2 - y1, 0.0)
            tgt_area = jnp.maximum(tx2 - tx1, 0.0) * jnp.maximum(ty2 - ty1, 0.0)
            inter = jnp.maximum(jnp.minimum(x2, tx2) - jnp.maximum(x1, tx1), 0.0) * \
                jnp.maximum(jnp.minimum(y2, ty2) - jnp.maximum(y1, ty1), 0.0)
            union_e = pred_area + tgt_area - inter + _EPS
            sx = (x1 + x2) - (tx1 + tx2)
            sy = (y1 + y2) - (ty1 + ty2)
            cdist = 0.25 * (sx * sx + sy * sy)
            ew = jnp.maximum(x2, tx2) - jnp.minimum(x1, tx1)
            eh = jnp.maximum(y2, ty2) - jnp.minimum(y1, ty1)
            diag_e = ew * ew + eh * eh + _EPS
            a_d = a_d + (cdist * union_e - inter * diag_e) / (union_e * diag_e)

            for p, t in ((x1, tx1), (y1, ty1), (x2, tx2), (y2, ty2)):
                ad = jnp.abs(p - t)
                cc = jnp.minimum(ad, 1.0)
                a_sl = a_sl + (ad - cc)
                a_sq = a_sq + cc * cc

            xd = x1 - x2
            yd = y1 - y2
            a_p = a_p + jnp.maximum(xd, 0.0) + jnp.maximum(yd, 0.0)
            a_p = a_p + jnp.where(xd >= 1.0, 1.0, 0.0)
            a_p = a_p + jnp.where(yd >= 1.0, 1.0, 0.0)
            for v in (x1, y1, x2, y2):
                a_n = a_n + jnp.minimum(v, 0.0)
                a_e = a_e + jnp.maximum(v, _DESIRED_SIZE)

        return a_d, a_sl, a_sq, a_p, a_n, a_e

    a_d, a_sl, a_sq, a_p, a_n, a_e = _acc
    acc_v[0, :] = a_d
    acc_v[1, :] = a_sl
    acc_v[2, :] = a_sq
    acc_v[3, :] = a_p
    acc_v[4, :] = a_n
    acc_v[5, :] = a_e - (4.0 * _DESIRED_SIZE * _GROUPS)
    acc_v[6, :] = zeros
    acc_v[7, :] = zeros
    pltpu.sync_copy(acc_v, out_hbm.at[wid])


def _sums6(x1, y1, x2, y2, tx1, ty1, tx2, ty2):
    pred_area = jnp.maximum(x2 - x1, 0.0) * jnp.maximum(y2 - y1, 0.0)
    tgt_area = jnp.maximum(tx2 - tx1, 0.0) * jnp.maximum(ty2 - ty1, 0.0)
    inter = jnp.maximum(jnp.minimum(x2, tx2) - jnp.maximum(x1, tx1), 0.0) * \
        jnp.maximum(jnp.minimum(y2, ty2) - jnp.maximum(y1, ty1), 0.0)
    union_e = pred_area + tgt_area - inter + _EPS
    sx = (x1 + x2) - (tx1 + tx2)
    sy = (y1 + y2) - (ty1 + ty2)
    cdist = 0.25 * (sx * sx + sy * sy)
    ew = jnp.maximum(x2, tx2) - jnp.minimum(x1, tx1)
    eh = jnp.maximum(y2, ty2) - jnp.minimum(y1, ty1)
    diag_e = ew * ew + eh * eh + _EPS
    s0 = jnp.sum((cdist * union_e - inter * diag_e) / (union_e * diag_e))
    s1 = 0.0
    s2 = 0.0
    for p, t in ((x1, tx1), (y1, ty1), (x2, tx2), (y2, ty2)):
        ad = jnp.abs(p - t)
        cc = jnp.minimum(ad, 1.0)
        s1 = s1 + jnp.sum(ad - cc)
        s2 = s2 + jnp.sum(cc * cc)
    xd = x1 - x2
    yd = y1 - y2
    s3 = jnp.sum(jnp.maximum(xd, 0.0) + jnp.maximum(yd, 0.0)
                 + jnp.where(xd >= 1.0, 1.0, 0.0)
                 + jnp.where(yd >= 1.0, 1.0, 0.0))
    s4 = 0.0
    s5 = 0.0
    for v in (x1, y1, x2, y2):
        s4 = s4 + jnp.sum(jnp.minimum(v, 0.0))
        s5 = s5 + jnp.sum(jnp.maximum(v - _DESIRED_SIZE, 0.0))
    return jnp.stack([s0, s1, s2, s3, s4, s5, 0.0, 0.0])


def kernel(pred_boxes, target_boxes):
    pred_cols = jnp.transpose(pred_boxes, (2, 0, 1))
    tgt_cols = jnp.transpose(target_boxes, (0, 2, 1))
    parts = _loss_partials(pred_cols, tgt_cols)
    tp = pred_boxes[:, _CM:, :]
    tt = target_boxes[:, _CM:, :]
    tail = _sums6(tp[..., 0], tp[..., 1], tp[..., 2], tp[..., 3],
                  tt[..., 0], tt[..., 1], tt[..., 2], tt[..., 3])
    s = jnp.sum(parts, axis=(0, 2)) + tail
    dl = 1.0 + s[0] / _N_BOXES
    sl = (s[1] + 0.5 * s[2]) / (_N_BOXES * 4)
    pen = (s[3] - s[4] + s[5]) / _DESIRED_SIZE
    return _ALPHA * dl + (1.0 - _ALPHA) * sl + _PENALTY_WEIGHT * pen


# speedup vs baseline: 1.2258x; 1.0008x over previous
import functools

import jax
import jax.numpy as jnp
from jax import lax
from jax.experimental import pallas as pl
from jax.experimental.pallas import tpu as pltpu
from jax.experimental.pallas import tpu_sc as plsc

_DESIRED_SIZE = 448.0
_ALPHA = 0.5
_PENALTY_WEIGHT = 0.015
_EPS = 1e-7

_NC = 2
_NS = 16
_NW = _NC * _NS
_L = 16

_B = 16
_R = 20000
_N_BOXES = _B * _R
_CM = 19968
_CPW = _CM // 16
_DMA_COLS = 1280
_GROUPS = 8 * (_CPW // _L)

_mesh = plsc.VectorSubcoreMesh(
    core_axis_name="c", subcore_axis_name="s", num_cores=_NC, num_subcores=_NS
)


@functools.partial(
    pl.kernel,
    out_type=jax.ShapeDtypeStruct((_NW, 8, _L), jnp.float32),
    mesh=_mesh,
    scratch_types=[
        pltpu.VMEM((8, _DMA_COLS), jnp.float32),
        pltpu.VMEM((8, _DMA_COLS), jnp.float32),
        pltpu.VMEM((8, _DMA_COLS), jnp.float32),
        pltpu.VMEM((8, _DMA_COLS), jnp.float32),
        pltpu.VMEM((8, 4, _DMA_COLS), jnp.float32),
        pltpu.VMEM((8, _L), jnp.float32),
    ],
    compiler_params=pltpu.CompilerParams(
        needs_layout_passes=False, use_tc_tiling_on_sc=True,
        skip_device_barrier=True),
)
def _loss_partials(pred_hbm, tgt_hbm, out_hbm,
                   x1v, y1v, x2v, y2v, tgt_v, acc_v):
    wid = lax.axis_index("s") * _NC + lax.axis_index("c")
    o = wid % 2
    k = wid // 2
    c0 = k * _CPW
    o0 = (c0 // 128) * 128
    rel0 = c0 - o0
    for c, dst in enumerate((x1v, y1v, x2v, y2v)):
        pltpu.sync_copy(
            pred_hbm.at[c, pl.ds(8 * o, 8), pl.ds(o0, _DMA_COLS)], dst)
    for b in range(8):
        pltpu.sync_copy(
            tgt_hbm.at[8 * o + b, pl.ds(0, 4), pl.ds(o0, _DMA_COLS)],
            tgt_v.at[b])

    zeros = jnp.zeros((_L,), jnp.float32)
    init = (zeros, zeros, zeros, zeros, zeros, zeros)

    @plsc.parallel_loop(0, _CPW // _L, 1, unroll=2, carry=init)
    def _acc(j, carry):
        a_d, a_sl, a_sq, a_p, a_n, a_e = carry
        col = rel0 + j * _L
        for b in range(8):
            x1 = x1v[b, pl.ds(col, _L)]
            y1 = y1v[b, pl.ds(col, _L)]
            x2 = x2v[b, pl.ds(col, _L)]
            y2 = y2v[b, pl.ds(col, _L)]
            tx1 = tgt_v[b, 0, pl.ds(col, _L)]
            ty1 = tgt_v[b, 1, pl.ds(col, _L)]
            tx2 = tgt_v[b, 2, pl.ds(col, _L)]
            ty2 = tgt_v[b, 3, pl.ds(col, _L)]

            pred_area = jnp.maximum(x2 - x1, 0.0) * jnp.maximum(y2 - y1, 0.0)
            tgt_area = jnp.maximum(tx2 - tx1, 0.0) * jnp.maximum(ty2 - ty1, 0.0)
            inter = jnp.maximum(jnp.minimum(x2, tx2) - jnp.maximum(x1, tx1), 0.0) * \
                jnp.maximum(jnp.minimum(y2, ty2) - jnp.maximum(y1, ty1), 0.0)
            union_e = pred_area + tgt_area - inter + _EPS
            sx = (x1 + x2) - (tx1 + tx2)
            sy = (y1 + y2) - (ty1 + ty2)
            cdist = 0.25 * (sx * sx + sy * sy)
            ew = jnp.maximum(x2, tx2) - jnp.minimum(x1, tx1)
            eh = jnp.maximum(y2, ty2) - jnp.minimum(y1, ty1)
            diag_e = ew * ew + eh * eh + _EPS
            a_d = a_d + (cdist * union_e - inter * diag_e) / (union_e * diag_e)

            for p, t in ((x1, tx1), (y1, ty1), (x2, tx2), (y2, ty2)):
                ad = jnp.abs(p - t)
                cc = jnp.minimum(ad, 1.0)
                a_sl = a_sl + (ad - cc)
                a_sq = a_sq + cc * cc

            xd = x1 - x2
            yd = y1 - y2
            a_p = a_p + jnp.maximum(xd, 0.0) + jnp.maximum(yd, 0.0)
            a_p = a_p + jnp.where(xd >= 1.0, 1.0, 0.0)
            a_p = a_p + jnp.where(yd >= 1.0, 1.0, 0.0)
            for v in (x1, y1, x2, y2):
                a_n = a_n + jnp.minimum(v, 0.0)
                a_e = a_e + jnp.maximum(v, _DESIRED_SIZE)

        return a_d, a_sl, a_sq, a_p, a_n, a_e

    a_d, a_sl, a_sq, a_p, a_n, a_e = _acc
    acc_v[0, :] = a_d
    acc_v[1, :] = a_sl
    acc_v[2, :] = a_sq
    acc_v[3, :] = a_p
    acc_v[4, :] = a_n
    acc_v[5, :] = a_e - (4.0 * _DESIRED_SIZE * _GROUPS)
    acc_v[6, :] = zeros
    acc_v[7, :] = zeros
    pltpu.sync_copy(acc_v, out_hbm.at[wid])


def _sums6(x1, y1, x2, y2, tx1, ty1, tx2, ty2):
    pred_area = jnp.maximum(x2 - x1, 0.0) * jnp.maximum(y2 - y1, 0.0)
    tgt_area = jnp.maximum(tx2 - tx1, 0.0) * jnp.maximum(ty2 - ty1, 0.0)
    inter = jnp.maximum(jnp.minimum(x2, tx2) - jnp.maximum(x1, tx1), 0.0) * \
        jnp.maximum(jnp.minimum(y2, ty2) - jnp.maximum(y1, ty1), 0.0)
    union_e = pred_area + tgt_area - inter + _EPS
    sx = (x1 + x2) - (tx1 + tx2)
    sy = (y1 + y2) - (ty1 + ty2)
    cdist = 0.25 * (sx * sx + sy * sy)
    ew = jnp.maximum(x2, tx2) - jnp.minimum(x1, tx1)
    eh = jnp.maximum(y2, ty2) - jnp.minimum(y1, ty1)
    diag_e = ew * ew + eh * eh + _EPS
    s0 = jnp.sum((cdist * union_e - inter * diag_e) / (union_e * diag_e))
    s1 = 0.0
    s2 = 0.0
    for p, t in ((x1, tx1), (y1, ty1), (x2, tx2), (y2, ty2)):
        ad = jnp.abs(p - t)
        cc = jnp.minimum(ad, 1.0)
        s1 = s1 + jnp.sum(ad - cc)
        s2 = s2 + jnp.sum(cc * cc)
    xd = x1 - x2
    yd = y1 - y2
    s3 = jnp.sum(jnp.maximum(xd, 0.0) + jnp.maximum(yd, 0.0)
                 + jnp.where(xd >= 1.0, 1.0, 0.0)
                 + jnp.where(yd >= 1.0, 1.0, 0.0))
    s4 = 0.0
    s5 = 0.0
    for v in (x1, y1, x2, y2):
        s4 = s4 + jnp.sum(jnp.minimum(v, 0.0))
        s5 = s5 + jnp.sum(jnp.maximum(v - _DESIRED_SIZE, 0.0))
    return jnp.stack([s0, s1, s2, s3, s4, s5, 0.0, 0.0])


def kernel(pred_boxes, target_boxes):
    pred_cols = jnp.transpose(pred_boxes, (2, 0, 1))
    tgt_cols = jnp.transpose(target_boxes, (0, 2, 1))
    parts = _loss_partials(pred_cols, tgt_cols)
    tp = pred_boxes[:, _CM:, :]
    tt = target_boxes[:, _CM:, :]
    tail = _sums6(tp[..., 0], tp[..., 1], tp[..., 2], tp[..., 3],
                  tt[..., 0], tt[..., 1], tt[..., 2], tt[..., 3])
    s = jnp.sum(parts, axis=(0, 2)) + tail
    dl = 1.0 + s[0] / _N_BOXES
    sl = (s[1] + 0.5 * s[2]) / (_N_BOXES * 4)
    pen = (s[3] - s[4] + s[5]) / _DESIRED_SIZE
    return _ALPHA * dl + (1.0 - _ALPHA) * sl + _PENALTY_WEIGHT * pen
